# baseline (speedup 1.0000x reference)
import jax
import jax.numpy as jnp
from jax.experimental import pallas as pl
from jax.experimental.pallas import tpu as pltpu

BUCKET_W = 4.0
N_ROWS = 1_000_000
D = 64
BC = 16384
GRID_A = -(-N_ROWS // BC)
BS = 131072
GRID_B = -(-N_ROWS // BS)


def _proj_body(x_ref, a_ref, b_ref, hash_ref, min_ref, min_sc):
    i = pl.program_id(0)
    b = b_ref[0]
    proj = jax.lax.dot_general(
        a_ref[...], x_ref[...],
        dimension_numbers=(((1,), (0,)), ((), ())),
        preferred_element_type=jnp.float32,
    )
    h = jnp.floor((proj + b) * (1.0 / BUCKET_W))
    cols = i * BC + jax.lax.broadcasted_iota(jnp.int32, (1, BC), 1)
    hmin = jnp.min(jnp.where(cols < N_ROWS, h, jnp.inf))

    @pl.when(i == 0)
    def _():
        min_sc[0] = hmin

    @pl.when(i > 0)
    def _():
        min_sc[0] = jnp.minimum(min_sc[0], hmin)

    hash_ref[...] = h.reshape(BC).astype(jnp.int16)

    @pl.when(i == GRID_A - 1)
    def _():
        min_ref[0] = min_sc[0].astype(jnp.int32)


def _sub_body(h_ref, m_ref, o_ref):
    o_ref[...] = h_ref[...].astype(jnp.int32) - m_ref[0]


def kernel(inputs, a, b):
    xt = inputs.T
    a2 = a.reshape(1, D)
    hash_u, minv = pl.pallas_call(
        _proj_body,
        grid=(GRID_A,),
        in_specs=[
            pl.BlockSpec((D, BC), lambda i: (0, i)),
            pl.BlockSpec((1, D), lambda i: (0, 0)),
            pl.BlockSpec(memory_space=pltpu.SMEM),
        ],
        out_specs=[
            pl.BlockSpec((BC,), lambda i: (i,)),
            pl.BlockSpec(memory_space=pltpu.SMEM),
        ],
        out_shape=[
            jax.ShapeDtypeStruct((N_ROWS,), jnp.int16),
            jax.ShapeDtypeStruct((1,), jnp.int32),
        ],
        scratch_shapes=[pltpu.SMEM((1,), jnp.float32)],
    )(xt, a2, b)

    out = pl.pallas_call(
        _sub_body,
        grid=(GRID_B,),
        in_specs=[
            pl.BlockSpec((BS,), lambda i: (i,)),
            pl.BlockSpec(memory_space=pltpu.SMEM),
        ],
        out_specs=pl.BlockSpec((BS,), lambda i: (i,)),
        out_shape=jax.ShapeDtypeStruct((N_ROWS,), jnp.int32),
    )(hash_u, minv)
    return out


# speedup vs baseline: 1.2165x; 1.2165x over previous
import jax
import jax.numpy as jnp
from jax.experimental import pallas as pl
from jax.experimental.pallas import tpu as pltpu

BUCKET_W = 4.0
N_ROWS = 1_000_000
D = 64
BC = 32768
GRID_A = -(-N_ROWS // BC)
BS = 131072
GRID_B = -(-N_ROWS // BS)


def _proj_body(x_ref, a_ref, b_ref, hash_ref, min_ref, min_sc):
    i = pl.program_id(0)
    b = b_ref[0]
    proj = jax.lax.dot_general(
        a_ref[...], x_ref[...],
        dimension_numbers=(((1,), (0,)), ((), ())),
        preferred_element_type=jnp.float32,
    )
    h = jnp.floor((proj + b) * (1.0 / BUCKET_W))
    cols = i * BC + jax.lax.broadcasted_iota(jnp.int32, (1, BC), 1)
    hmin = jnp.min(jnp.where(cols < N_ROWS, h, jnp.inf))

    @pl.when(i == 0)
    def _():
        min_sc[0] = hmin

    @pl.when(i > 0)
    def _():
        min_sc[0] = jnp.minimum(min_sc[0], hmin)

    hash_ref[...] = h.reshape(BC).astype(jnp.int32)

    @pl.when(i == GRID_A - 1)
    def _():
        min_ref[0] = min_sc[0].astype(jnp.int32)


def _sub_body(h_ref, m_ref, o_ref):
    o_ref[...] = h_ref[...] - m_ref[0]


def kernel(inputs, a, b):
    xt = inputs.T
    a2 = a.reshape(1, D)
    hash_u, minv = pl.pallas_call(
        _proj_body,
        grid=(GRID_A,),
        in_specs=[
            pl.BlockSpec((D, BC), lambda i: (0, i)),
            pl.BlockSpec((1, D), lambda i: (0, 0)),
            pl.BlockSpec(memory_space=pltpu.SMEM),
        ],
        out_specs=[
            pl.BlockSpec((BC,), lambda i: (i,)),
            pl.BlockSpec(memory_space=pltpu.SMEM),
        ],
        out_shape=[
            jax.ShapeDtypeStruct((N_ROWS,), jnp.int32),
            jax.ShapeDtypeStruct((1,), jnp.int32),
        ],
        scratch_shapes=[pltpu.SMEM((1,), jnp.float32)],
    )(xt, a2, b)

    out = pl.pallas_call(
        _sub_body,
        grid=(GRID_B,),
        in_specs=[
            pl.BlockSpec((BS,), lambda i: (i,)),
            pl.BlockSpec(memory_space=pltpu.SMEM),
        ],
        out_specs=pl.BlockSpec((BS,), lambda i: (i,)),
        out_shape=jax.ShapeDtypeStruct((N_ROWS,), jnp.int32),
    )(hash_u, minv)
    return out
